# baseline (speedup 1.0000x reference)
import functools

import jax
import jax.numpy as jnp
from jax import lax
from jax.experimental import pallas as pl
from jax.experimental.pallas import tpu as pltpu
from jax.experimental.pallas import tpu_sc as plsc

VOCAB, B, S, H = 30522, 128, 512, 768
NC, NS = 2, 16
NW = NC * NS
NSLICE = 1
BSL = B // NSLICE
TOK_PER_W = BSL * S // NW
GK = 64
NG = TOK_PER_W // GK
NBUF = 2
EPS = 1e-12


def _gather_body(ids_hbm, wemb_hbm, out_hbm, *scratch):
    cid = lax.axis_index("c")
    sid = lax.axis_index("s")
    wid = sid * NC + cid
    base = wid * TOK_PER_W

    idx = scratch[0:NBUF]
    buf = scratch[NBUF:2 * NBUF]
    gsem = scratch[2 * NBUF:3 * NBUF]
    osem = scratch[3 * NBUF:4 * NBUF]

    def start_gather(i, b):
        pltpu.sync_copy(ids_hbm.at[pl.ds(base + i * GK, GK)], idx[b])
        return pltpu.async_copy(wemb_hbm.at[idx[b]], buf[b], gsem[b])

    gathers = [None] * NBUF
    outs = [None] * NBUF
    for i in range(NBUF - 1):
        gathers[i] = start_gather(i, i)
    for i in range(NG):
        b = i % NBUF
        gathers[b].wait()
        nxt = i + NBUF - 1
        if nxt < NG:
            nb = nxt % NBUF
            if outs[nb] is not None:
                outs[nb].wait()
            gathers[nb] = start_gather(nxt, nb)
        outs[b] = pltpu.async_copy(
            buf[b], out_hbm.at[pl.ds(base + i * GK, GK)], osem[b])
    for j in range(NBUF):
        outs[j].wait()


_mesh = plsc.VectorSubcoreMesh(core_axis_name="c", subcore_axis_name="s",
                               num_cores=NC, num_subcores=NS)

_sc_gather = pl.kernel(
    _gather_body,
    out_type=jax.ShapeDtypeStruct((BSL * S, H), jnp.float32),
    mesh=_mesh,
    scratch_types=(
        [pltpu.VMEM((GK,), jnp.int32)] * NBUF
        + [pltpu.VMEM((GK, H), jnp.float32)] * NBUF
        + [pltpu.SemaphoreType.DMA] * (2 * NBUF)
    ),
)


RB = 8


def _ln_body(w_ref, tt_ref, pos_ref, type_ref, gamma_ref, beta_ref, out_ref):
    for r in range(RB):
        w = w_ref[r]
        tt = tt_ref[r]
        pos = pos_ref[...]
        tsel = jnp.where((tt[0][:, None]) == 1, type_ref[1][None, :],
                         type_ref[0][None, :])
        v = w + pos + tsel
        mean = jnp.mean(v, axis=-1, keepdims=True)
        c = v - mean
        var = jnp.mean(c * c, axis=-1, keepdims=True)
        normed = c * jax.lax.rsqrt(var + EPS)
        out_ref[r] = normed * gamma_ref[...][None, :] + beta_ref[...][None, :]


_tc_ln = pl.pallas_call(
    _ln_body,
    grid=(BSL // RB,),
    in_specs=[
        pl.BlockSpec((RB, S, H), lambda i: (i, 0, 0)),
        pl.BlockSpec((RB, 1, S), lambda i: (i, 0, 0)),
        pl.BlockSpec((S, H), lambda i: (0, 0)),
        pl.BlockSpec((2, H), lambda i: (0, 0)),
        pl.BlockSpec((H,), lambda i: (0,)),
        pl.BlockSpec((H,), lambda i: (0,)),
    ],
    out_specs=pl.BlockSpec((RB, S, H), lambda i: (i, 0, 0)),
    out_shape=jax.ShapeDtypeStruct((BSL, S, H), jnp.float32),
)


@jax.jit
def kernel(input_ids, token_type_ids, word_emb, pos_emb, type_emb, gamma, beta):
    ids = input_ids.astype(jnp.int32).reshape(NSLICE, BSL * S)
    tts = token_type_ids.astype(jnp.int32).reshape(NSLICE, BSL, 1, S)
    outs = []
    for sl in range(NSLICE):
        gathered = _sc_gather(ids[sl], word_emb).reshape(BSL, S, H)
        outs.append(_tc_ln(gathered, tts[sl], pos_emb, type_emb, gamma, beta))
    return jnp.concatenate(outs, axis=0)


# speedup vs baseline: 1.0173x; 1.0173x over previous
import functools

import jax
import jax.numpy as jnp
from jax import lax
from jax.experimental import pallas as pl
from jax.experimental.pallas import tpu as pltpu
from jax.experimental.pallas import tpu_sc as plsc

VOCAB, B, S, H = 30522, 128, 512, 768
NC, NS = 2, 16
NW = NC * NS
NSLICE = 1
BSL = B // NSLICE
TOK_PER_W = BSL * S // NW
GK = 64
NG = TOK_PER_W // GK
NBUF = 2
EPS = 1e-12


def _gather_body(ids_hbm, wemb_hbm, out_hbm, *scratch):
    cid = lax.axis_index("c")
    sid = lax.axis_index("s")
    wid = sid * NC + cid
    base = wid * TOK_PER_W

    idall = scratch[0]
    buf = scratch[1:1 + NBUF]
    gsem = scratch[1 + NBUF:1 + 2 * NBUF]
    osem = scratch[1 + 2 * NBUF:1 + 3 * NBUF]

    pltpu.sync_copy(ids_hbm.at[pl.ds(base, TOK_PER_W)], idall)

    def start_gather(i, b):
        return pltpu.async_copy(
            wemb_hbm.at[idall.at[pl.ds(i * GK, GK)]], buf[b], gsem[b])

    gathers = [None] * NBUF
    outs = [None] * NBUF
    for i in range(NBUF - 1):
        gathers[i] = start_gather(i, i)
    for i in range(NG):
        b = i % NBUF
        gathers[b].wait()
        nxt = i + NBUF - 1
        if nxt < NG:
            nb = nxt % NBUF
            if outs[nb] is not None:
                outs[nb].wait()
            gathers[nb] = start_gather(nxt, nb)
        outs[b] = pltpu.async_copy(
            buf[b], out_hbm.at[pl.ds(base + i * GK, GK)], osem[b])
    for j in range(NBUF):
        outs[j].wait()


_mesh = plsc.VectorSubcoreMesh(core_axis_name="c", subcore_axis_name="s",
                               num_cores=NC, num_subcores=NS)

_sc_gather = pl.kernel(
    _gather_body,
    out_type=jax.ShapeDtypeStruct((BSL * S, H), jnp.float32),
    mesh=_mesh,
    scratch_types=(
        [pltpu.VMEM((TOK_PER_W,), jnp.int32)]
        + [pltpu.VMEM((GK, H), jnp.float32)] * NBUF
        + [pltpu.SemaphoreType.DMA] * (2 * NBUF)
    ),
)


RB = 8


def _ln_body(w_ref, tt_ref, pos_ref, type_ref, gamma_ref, beta_ref, out_ref):
    for r in range(RB):
        w = w_ref[r]
        tt = tt_ref[r]
        pos = pos_ref[...]
        tsel = jnp.where((tt[0][:, None]) == 1, type_ref[1][None, :],
                         type_ref[0][None, :])
        v = w + pos + tsel
        mean = jnp.mean(v, axis=-1, keepdims=True)
        c = v - mean
        var = jnp.mean(c * c, axis=-1, keepdims=True)
        normed = c * jax.lax.rsqrt(var + EPS)
        out_ref[r] = normed * gamma_ref[...][None, :] + beta_ref[...][None, :]


_tc_ln = pl.pallas_call(
    _ln_body,
    grid=(BSL // RB,),
    in_specs=[
        pl.BlockSpec((RB, S, H), lambda i: (i, 0, 0)),
        pl.BlockSpec((RB, 1, S), lambda i: (i, 0, 0)),
        pl.BlockSpec((S, H), lambda i: (0, 0)),
        pl.BlockSpec((2, H), lambda i: (0, 0)),
        pl.BlockSpec((H,), lambda i: (0,)),
        pl.BlockSpec((H,), lambda i: (0,)),
    ],
    out_specs=pl.BlockSpec((RB, S, H), lambda i: (i, 0, 0)),
    out_shape=jax.ShapeDtypeStruct((BSL, S, H), jnp.float32),
)


@jax.jit
def kernel(input_ids, token_type_ids, word_emb, pos_emb, type_emb, gamma, beta):
    ids = input_ids.astype(jnp.int32).reshape(NSLICE, BSL * S)
    tts = token_type_ids.astype(jnp.int32).reshape(NSLICE, BSL, 1, S)
    outs = []
    for sl in range(NSLICE):
        gathered = _sc_gather(ids[sl], word_emb).reshape(BSL, S, H)
        outs.append(_tc_ln(gathered, tts[sl], pos_emb, type_emb, gamma, beta))
    return jnp.concatenate(outs, axis=0)
